# speedup vs baseline: 2.7065x; 2.7065x over previous
import functools

import jax
import jax.numpy as jnp
from jax import lax
from jax.experimental import pallas as pl
from jax.experimental.pallas import tpu as pltpu
from jax.experimental.pallas import tpu_sc as plsc

_N = 1000000
_B = 16384
_DIH_EPS = 0.076
_GNN_EPS = 6.38
_FS_EPS = 3.0
_LR = 0.001

_NS = 16
_PB = _B // _NS
_RB = _PB // 128
_COPY = 62496
_TAIL = _N - _NS * _COPY


def _sc_body(idx_hbm, dih_hbm, gnn_hbm, fs_hbm, lamd_hbm, lamg_hbm, lamf_hbm,
             outd_hbm, outg_hbm, outf_hbm, part_hbm,
             cbuf, idx_v, loss_v, lam_v, new_v, pacc, sem):
  cid = lax.axis_index("c")
  sid = lax.axis_index("s")

  def copy_buf(src, dst):
    off = sid * _COPY
    pltpu.sync_copy(src.at[pl.ds(off, _COPY)], cbuf)
    pltpu.sync_copy(cbuf, dst.at[pl.ds(off, _COPY)])

    @pl.when(sid == 0)
    def _():
      pltpu.sync_copy(src.at[pl.ds(_NS * _COPY, _TAIL)], cbuf.at[pl.ds(0, _TAIL)])
      pltpu.sync_copy(cbuf.at[pl.ds(0, _TAIL)], dst.at[pl.ds(_NS * _COPY, _TAIL)])

  @pl.when(cid == 0)
  def _():
    copy_buf(lamd_hbm, outd_hbm)

  @pl.when(cid == 1)
  def _():
    copy_buf(lamg_hbm, outg_hbm)
    copy_buf(lamf_hbm, outf_hbm)

  plsc.subcore_barrier()

  pltpu.sync_copy(idx_hbm.at[sid], idx_v)

  def process(loss_hbm, lam_hbm, out_hbm, eps):
    pltpu.sync_copy(loss_hbm.at[sid], loss_v)
    cps = [pltpu.async_copy(lam_hbm.at[idx_v.at[j]], lam_v.at[j], sem)
           for j in range(_RB)]
    for cp in cps:
      cp.wait()
    acc = jnp.zeros((16,), jnp.float32)
    for j in range(_RB):
      for k in range(8):
        lam = lam_v[j, pl.ds(k * 16, 16)]
        viol = loss_v[j, pl.ds(k * 16, 16)] - eps
        acc = acc + lam * viol
        new_v[j, pl.ds(k * 16, 16)] = jnp.maximum(lam + _LR * viol, 0.0)
    cps = [pltpu.async_copy(new_v.at[j], out_hbm.at[idx_v.at[j]], sem)
           for j in range(_RB)]
    for cp in cps:
      cp.wait()
    return acc

  zero = jnp.zeros((16,), jnp.float32)

  @pl.when(cid == 0)
  def _():
    acc_d = process(dih_hbm, lamd_hbm, outd_hbm, _DIH_EPS)
    pacc[0, pl.ds(0, 16)] = acc_d
    pacc[1, pl.ds(0, 16)] = zero
    pacc[2, pl.ds(0, 16)] = zero

  @pl.when(cid == 1)
  def _():
    acc_g = process(gnn_hbm, lamg_hbm, outg_hbm, _GNN_EPS)
    acc_f = process(fs_hbm, lamf_hbm, outf_hbm, _FS_EPS)
    pacc[0, pl.ds(0, 16)] = zero
    pacc[1, pl.ds(0, 16)] = acc_g
    pacc[2, pl.ds(0, 16)] = acc_f

  pltpu.sync_copy(pacc, part_hbm.at[cid, sid])


_sc_call = functools.partial(
    pl.kernel,
    out_type=(
        jax.ShapeDtypeStruct((_N,), jnp.float32),
        jax.ShapeDtypeStruct((_N,), jnp.float32),
        jax.ShapeDtypeStruct((_N,), jnp.float32),
        jax.ShapeDtypeStruct((2, _NS, 3, 16), jnp.float32),
    ),
    mesh=plsc.VectorSubcoreMesh(core_axis_name="c", subcore_axis_name="s"),
    scratch_types=[
        pltpu.VMEM((_COPY,), jnp.float32),
        pltpu.VMEM((_RB, 128), jnp.int32),
        pltpu.VMEM((_RB, 128), jnp.float32),
        pltpu.VMEM((_RB, 128), jnp.float32),
        pltpu.VMEM((_RB, 128), jnp.float32),
        pltpu.VMEM((3, 16), jnp.float32),
        pltpu.SemaphoreType.DMA,
    ],
)(_sc_body)


def kernel(primary_loss, dihedral_losses, gnn_losses, foldseek_losses, indices,
           lam_dihedral, lam_gnn, lam_foldseek):
  idx3 = indices.astype(jnp.int32).reshape(_NS, _RB, 128)
  dih3 = dihedral_losses.reshape(_NS, _RB, 128)
  gnn3 = gnn_losses.reshape(_NS, _RB, 128)
  fs3 = foldseek_losses.reshape(_NS, _RB, 128)
  out_d, out_g, out_f, part = _sc_call(
      idx3, dih3, gnn3, fs3, lam_dihedral, lam_gnn, lam_foldseek)
  lagrangian = primary_loss + jnp.sum(part) / jnp.float32(_B)
  return lagrangian, out_d, out_g, out_f
